# Initial kernel scaffold; baseline (speedup 1.0000x reference)
import jax
import jax.numpy as jnp
from jax.experimental import pallas as pl


def kernel(x, edge_index, W0a, b0a, W0b, b0b, W1a, b1a, W1b, b1b, Wc1, bc1, Wc2, bc2):
    raise NotImplementedError("write your pallas kernel here")


# speedup vs baseline: 3.3886x; 3.3886x over previous
import functools

import jax
import jax.numpy as jnp
from jax import lax
from jax.experimental import pallas as pl
from jax.experimental.pallas import tpu as pltpu
from jax.experimental.pallas import tpu_sc as plsc

N = 10000
D = 128
E = 320000
NC = 2
NS = 16
NW = NC * NS
EW = E // NW
C = 128
K = 80
EW_PAD = K * C
RPW = 632
ACC_ROWS = NS * RPW
TRASH = N
BM = 1000
GRID = N // BM


def _leaky(v):
    return jnp.where(v > 0, v, 0.01 * v)


def _segment_sum_sc(x, src_r, dst_r, zeros):
    mesh = plsc.VectorSubcoreMesh(
        core_axis_name="c", subcore_axis_name="s", num_cores=NC, num_subcores=NS
    )

    @functools.partial(
        pl.kernel,
        out_type=jax.ShapeDtypeStruct((2 * ACC_ROWS, D), jnp.float32),
        mesh=mesh,
        scratch_types=[
            pltpu.VMEM((K, C), jnp.int32),
            pltpu.VMEM((K, C), jnp.int32),
            pltpu.VMEM((C, D), jnp.float32),
            pltpu.VMEM_SHARED((ACC_ROWS, D), jnp.float32),
        ],
    )
    def segsum(x_hbm, src_hbm, dst_hbm, z_hbm, out_hbm, idx_s, idx_d, rows, acc):
        cid = lax.axis_index("c")
        sid = lax.axis_index("s")
        wid = cid * NS + sid

        pltpu.sync_copy(src_hbm.at[wid], idx_s)
        pltpu.sync_copy(dst_hbm.at[wid], idx_d)

        pltpu.sync_copy(z_hbm, acc.at[pl.ds(sid * RPW, RPW)])

        plsc.subcore_barrier()

        @pl.loop(0, K)
        def _(j):
            pltpu.sync_copy(x_hbm.at[idx_s.at[j]], rows)
            pltpu.sync_copy(rows, acc.at[idx_d.at[j]], add=True)

        plsc.subcore_barrier()

        pltpu.sync_copy(acc.at[pl.ds(sid * RPW, RPW)],
                        out_hbm.at[pl.ds(cid * ACC_ROWS + sid * RPW, RPW)])

    return segsum(x, src_r, dst_r, zeros)


def _mlp_body(x_ref, p0_ref, p1_ref, wa_ref, ba_ref, wb_ref, bb_ref, o_ref):
    pre = 1.1 * x_ref[...] + p0_ref[0] + p1_ref[0]
    t = _leaky(
        jnp.dot(pre, wa_ref[...], preferred_element_type=jnp.float32,
                precision=lax.Precision.HIGHEST) + ba_ref[...]
    )
    v = jnp.dot(t, wb_ref[...], preferred_element_type=jnp.float32,
                precision=lax.Precision.HIGHEST) + bb_ref[...]
    o_ref[...] = _leaky(_leaky(v))


def _mlp_layer(x, p, wa, ba, wb, bb):
    row = lambda i: (i, 0)
    full = lambda i: (0, 0)
    return pl.pallas_call(
        _mlp_body,
        out_shape=jax.ShapeDtypeStruct((N, D), jnp.float32),
        grid=(GRID,),
        in_specs=[
            pl.BlockSpec((BM, D), row),
            pl.BlockSpec((1, BM, D), lambda i: (0, i, 0)),
            pl.BlockSpec((1, BM, D), lambda i: (1, i, 0)),
            pl.BlockSpec((D, D), full),
            pl.BlockSpec((1, D), full),
            pl.BlockSpec((D, D), full),
            pl.BlockSpec((1, D), full),
        ],
        out_specs=pl.BlockSpec((BM, D), row),
    )(x, p, p, wa, ba, wb, bb)


def _mlp2_body(x_ref, p0_ref, p1_ref, wa_ref, ba_ref, wb_ref, bb_ref,
               wc1_ref, bc1_ref, wc2_ref, bc2_ref, o_ref, acc_ref):
    i = pl.program_id(0)
    pre = 1.1 * x_ref[...] + p0_ref[0] + p1_ref[0]
    t = _leaky(
        jnp.dot(pre, wa_ref[...], preferred_element_type=jnp.float32,
                precision=lax.Precision.HIGHEST) + ba_ref[...]
    )
    v = jnp.dot(t, wb_ref[...], preferred_element_type=jnp.float32,
                precision=lax.Precision.HIGHEST) + bb_ref[...]
    h2 = _leaky(_leaky(v))
    s = jnp.sum(h2, axis=0, keepdims=True)

    @pl.when(i == 0)
    def _():
        acc_ref[...] = s

    @pl.when(i > 0)
    def _():
        acc_ref[...] += s

    @pl.when(i == pl.num_programs(0) - 1)
    def _():
        em = acc_ref[...]
        z = _leaky(
            jnp.dot(em, wc1_ref[...], preferred_element_type=jnp.float32,
                    precision=lax.Precision.HIGHEST) + bc1_ref[...]
        )
        o_ref[...] = jnp.dot(
            z, wc2_ref[...], preferred_element_type=jnp.float32,
            precision=lax.Precision.HIGHEST) + bc2_ref[...]


def _mlp_layer2(h, p, wa, ba, wb, bb, wc1, bc1, wc2p, bc2p):
    row = lambda i: (i, 0)
    full = lambda i: (0, 0)
    return pl.pallas_call(
        _mlp2_body,
        out_shape=jax.ShapeDtypeStruct((1, D), jnp.float32),
        grid=(GRID,),
        in_specs=[
            pl.BlockSpec((BM, D), row),
            pl.BlockSpec((1, BM, D), lambda i: (0, i, 0)),
            pl.BlockSpec((1, BM, D), lambda i: (1, i, 0)),
            pl.BlockSpec((D, D), full),
            pl.BlockSpec((1, D), full),
            pl.BlockSpec((D, D), full),
            pl.BlockSpec((1, D), full),
            pl.BlockSpec((D, D), full),
            pl.BlockSpec((1, D), full),
            pl.BlockSpec((D, D), full),
            pl.BlockSpec((1, D), full),
        ],
        out_specs=pl.BlockSpec((1, D), full),
        scratch_shapes=[pltpu.VMEM((1, D), jnp.float32)],
    )(h, p, p, wa, ba, wb, bb, wc1, bc1, wc2p, bc2p)


def kernel(x, edge_index, W0a, b0a, W0b, b0b, W1a, b1a, W1b, b1b,
           Wc1, bc1, Wc2, bc2):
    src = edge_index[0].astype(jnp.int32).reshape(NW, EW)
    dst = edge_index[1].astype(jnp.int32).reshape(NW, EW)
    src_r = jnp.pad(src, ((0, 0), (0, EW_PAD - EW))).reshape(NW, K, C)
    dst_r = jnp.pad(dst, ((0, 0), (0, EW_PAD - EW)),
                    constant_values=TRASH).reshape(NW, K, C)
    zeros = jnp.zeros((RPW, D), jnp.float32)

    b0a_, b0b_, b1a_, b1b_, bc1_ = (
        b.reshape(1, D) for b in (b0a, b0b, b1a, b1b, bc1))
    Wc2p = jnp.pad(Wc2, ((0, 0), (0, D - Wc2.shape[1])))
    bc2p = jnp.pad(bc2, (0, D - bc2.shape[0])).reshape(1, D)

    p1 = _segment_sum_sc(x, src_r, dst_r, zeros).reshape(2, ACC_ROWS, D)
    h1 = _mlp_layer(x, p1, W0a, b0a_, W0b, b0b_)
    p2 = _segment_sum_sc(h1, src_r, dst_r, zeros).reshape(2, ACC_ROWS, D)
    cl = _mlp_layer2(h1, p2, W1a, b1a_, W1b, b1b_, Wc1, bc1_, Wc2p, bc2p)
    return cl[:, : Wc2.shape[1]]


# speedup vs baseline: 3.6827x; 1.0868x over previous
import functools

import jax
import jax.numpy as jnp
from jax import lax
from jax.experimental import pallas as pl
from jax.experimental.pallas import tpu as pltpu
from jax.experimental.pallas import tpu_sc as plsc

N = 10000
D = 128
E = 320000
NC = 2
NS = 16
NW = NC * NS
EW = E // NW
C = 128
K = 80
KH = K // 2
EW_PAD = K * C
RPW = 632
ACC_ROWS = NS * RPW
TRASH = N
BM = 1000
GRID = N // BM


def _leaky(v):
    return jnp.where(v > 0, v, 0.01 * v)


def _segment_sum_sc(x, src_r, dst_r, zeros):
    mesh = plsc.VectorSubcoreMesh(
        core_axis_name="c", subcore_axis_name="s", num_cores=NC, num_subcores=NS
    )

    @functools.partial(
        pl.kernel,
        out_type=jax.ShapeDtypeStruct((2 * ACC_ROWS, D), jnp.float32),
        mesh=mesh,
        scratch_types=[
            pltpu.VMEM((KH, C), jnp.int32),
            pltpu.VMEM((KH, C), jnp.int32),
            pltpu.VMEM((C, D), jnp.float32),
            pltpu.VMEM((C, D), jnp.float32),
            pltpu.VMEM_SHARED((ACC_ROWS, D), jnp.float32),
            pltpu.SemaphoreType.DMA,
            pltpu.SemaphoreType.DMA,
        ],
    )
    def segsum(x_hbm, src_hbm, dst_hbm, z_hbm, out_hbm,
               idx_s, idx_d, r0, r1, acc, g0, g1):
        cid = lax.axis_index("c")
        sid = lax.axis_index("s")
        wid = cid * NS + sid

        pltpu.sync_copy(z_hbm, acc.at[pl.ds(sid * RPW, RPW)])

        plsc.subcore_barrier()

        def gather(j, rbuf, sem):
            return pltpu.async_copy(x_hbm.at[idx_s.at[j]], rbuf, sem)

        def gather_wait(rbuf, sem):
            pltpu.make_async_copy(x_hbm.at[pl.ds(0, C)], rbuf, sem).wait()

        def scatter_add(j, rbuf):
            pltpu.sync_copy(rbuf, acc.at[idx_d.at[j]], add=True)

        @pl.loop(0, 2)
        def _(h):
            pltpu.sync_copy(src_hbm.at[wid * 2 + h], idx_s)
            pltpu.sync_copy(dst_hbm.at[wid * 2 + h], idx_d)

            gather(0, r0, g0)

            @pl.loop(0, KH // 2 - 1)
            def _(p):
                j = 2 * p
                gather_wait(r0, g0)
                d1 = gather(j + 1, r1, g1)
                scatter_add(j, r0)
                d1.wait()
                gather(j + 2, r0, g0)
                scatter_add(j + 1, r1)

            gather_wait(r0, g0)
            d1 = gather(KH - 1, r1, g1)
            scatter_add(KH - 2, r0)
            d1.wait()
            scatter_add(KH - 1, r1)

        plsc.subcore_barrier()

        pltpu.sync_copy(acc.at[pl.ds(sid * RPW, RPW)],
                        out_hbm.at[pl.ds(cid * ACC_ROWS + sid * RPW, RPW)])

    return segsum(x, src_r, dst_r, zeros)


def _mlp_body(x_ref, p0_ref, p1_ref, wa_ref, ba_ref, wb_ref, bb_ref, o_ref):
    pre = 1.1 * x_ref[...] + p0_ref[0] + p1_ref[0]
    t = _leaky(
        jnp.dot(pre, wa_ref[...], preferred_element_type=jnp.float32,
                precision=lax.Precision.HIGHEST) + ba_ref[...]
    )
    v = jnp.dot(t, wb_ref[...], preferred_element_type=jnp.float32,
                precision=lax.Precision.HIGHEST) + bb_ref[...]
    o_ref[...] = _leaky(_leaky(v))


def _mlp_layer(x, p, wa, ba, wb, bb):
    row = lambda i: (i, 0)
    full = lambda i: (0, 0)
    return pl.pallas_call(
        _mlp_body,
        out_shape=jax.ShapeDtypeStruct((N, D), jnp.float32),
        grid=(GRID,),
        in_specs=[
            pl.BlockSpec((BM, D), row),
            pl.BlockSpec((1, BM, D), lambda i: (0, i, 0)),
            pl.BlockSpec((1, BM, D), lambda i: (1, i, 0)),
            pl.BlockSpec((D, D), full),
            pl.BlockSpec((1, D), full),
            pl.BlockSpec((D, D), full),
            pl.BlockSpec((1, D), full),
        ],
        out_specs=pl.BlockSpec((BM, D), row),
    )(x, p, p, wa, ba, wb, bb)


def _mlp2_body(x_ref, p0_ref, p1_ref, wa_ref, ba_ref, wb_ref, bb_ref,
               wc1_ref, bc1_ref, wc2_ref, bc2_ref, o_ref, acc_ref):
    i = pl.program_id(0)
    pre = 1.1 * x_ref[...] + p0_ref[0] + p1_ref[0]
    t = _leaky(
        jnp.dot(pre, wa_ref[...], preferred_element_type=jnp.float32,
                precision=lax.Precision.HIGHEST) + ba_ref[...]
    )
    v = jnp.dot(t, wb_ref[...], preferred_element_type=jnp.float32,
                precision=lax.Precision.HIGHEST) + bb_ref[...]
    h2 = _leaky(_leaky(v))
    s = jnp.sum(h2, axis=0, keepdims=True)

    @pl.when(i == 0)
    def _():
        acc_ref[...] = s

    @pl.when(i > 0)
    def _():
        acc_ref[...] += s

    @pl.when(i == pl.num_programs(0) - 1)
    def _():
        em = acc_ref[...]
        z = _leaky(
            jnp.dot(em, wc1_ref[...], preferred_element_type=jnp.float32,
                    precision=lax.Precision.HIGHEST) + bc1_ref[...]
        )
        o_ref[...] = jnp.dot(
            z, wc2_ref[...], preferred_element_type=jnp.float32,
            precision=lax.Precision.HIGHEST) + bc2_ref[...]


def _mlp_layer2(h, p, wa, ba, wb, bb, wc1, bc1, wc2p, bc2p):
    row = lambda i: (i, 0)
    full = lambda i: (0, 0)
    return pl.pallas_call(
        _mlp2_body,
        out_shape=jax.ShapeDtypeStruct((1, D), jnp.float32),
        grid=(GRID,),
        in_specs=[
            pl.BlockSpec((BM, D), row),
            pl.BlockSpec((1, BM, D), lambda i: (0, i, 0)),
            pl.BlockSpec((1, BM, D), lambda i: (1, i, 0)),
            pl.BlockSpec((D, D), full),
            pl.BlockSpec((1, D), full),
            pl.BlockSpec((D, D), full),
            pl.BlockSpec((1, D), full),
            pl.BlockSpec((D, D), full),
            pl.BlockSpec((1, D), full),
            pl.BlockSpec((D, D), full),
            pl.BlockSpec((1, D), full),
        ],
        out_specs=pl.BlockSpec((1, D), full),
        scratch_shapes=[pltpu.VMEM((1, D), jnp.float32)],
    )(h, p, p, wa, ba, wb, bb, wc1, bc1, wc2p, bc2p)


def kernel(x, edge_index, W0a, b0a, W0b, b0b, W1a, b1a, W1b, b1b,
           Wc1, bc1, Wc2, bc2):
    src = edge_index[0].astype(jnp.int32).reshape(NW, EW)
    dst = edge_index[1].astype(jnp.int32).reshape(NW, EW)
    src_r = jnp.pad(src, ((0, 0), (0, EW_PAD - EW))).reshape(NW * 2, KH, C)
    dst_r = jnp.pad(dst, ((0, 0), (0, EW_PAD - EW)),
                    constant_values=TRASH).reshape(NW * 2, KH, C)
    zeros = jnp.zeros((RPW, D), jnp.float32)

    b0a_, b0b_, b1a_, b1b_, bc1_ = (
        b.reshape(1, D) for b in (b0a, b0b, b1a, b1b, bc1))
    Wc2p = jnp.pad(Wc2, ((0, 0), (0, D - Wc2.shape[1])))
    bc2p = jnp.pad(bc2, (0, D - bc2.shape[0])).reshape(1, D)

    p1 = _segment_sum_sc(x, src_r, dst_r, zeros).reshape(2, ACC_ROWS, D)
    h1 = _mlp_layer(x, p1, W0a, b0a_, W0b, b0b_)
    p2 = _segment_sum_sc(h1, src_r, dst_r, zeros).reshape(2, ACC_ROWS, D)
    cl = _mlp_layer2(h1, p2, W1a, b1a_, W1b, b1b_, Wc1, bc1_, Wc2p, bc2p)
    return cl[:, : Wc2.shape[1]]


# speedup vs baseline: 3.7425x; 1.0162x over previous
import functools

import jax
import jax.numpy as jnp
from jax import lax
from jax.experimental import pallas as pl
from jax.experimental.pallas import tpu as pltpu
from jax.experimental.pallas import tpu_sc as plsc

N = 10000
D = 128
E = 320000
NC = 2
NS = 16
NW = NC * NS
EW = E // NW
C = 128
K = 80
KH = K // 2
EW_PAD = K * C
RPW = 632
ACC_ROWS = NS * RPW
TRASH = N
BM = 1000
GRID = N // BM


def _leaky(v):
    return jnp.where(v > 0, v, 0.01 * v)


def _segment_sum_sc(x, src_r, dst_r, zeros):
    mesh = plsc.VectorSubcoreMesh(
        core_axis_name="c", subcore_axis_name="s", num_cores=NC, num_subcores=NS
    )

    @functools.partial(
        pl.kernel,
        out_type=jax.ShapeDtypeStruct((2 * ACC_ROWS, D), jnp.float32),
        mesh=mesh,
        scratch_types=[
            pltpu.VMEM((KH, C), jnp.int32),
            pltpu.VMEM((KH, C), jnp.int32),
            pltpu.VMEM((C, D), jnp.float32),
            pltpu.VMEM((C, D), jnp.float32),
            pltpu.VMEM_SHARED((ACC_ROWS, D), jnp.float32),
            pltpu.SemaphoreType.DMA,
            pltpu.SemaphoreType.DMA,
        ],
    )
    def segsum(x_hbm, src_hbm, dst_hbm, z_hbm, out_hbm,
               idx_s, idx_d, r0, r1, acc, g0, g1):
        cid = lax.axis_index("c")
        sid = lax.axis_index("s")
        wid = cid * NS + sid

        pltpu.sync_copy(z_hbm, acc.at[pl.ds(sid * RPW, RPW)])

        plsc.subcore_barrier()

        def gather(j, rbuf, sem):
            return pltpu.async_copy(x_hbm.at[idx_s.at[j]], rbuf, sem)

        def gather_wait(rbuf, sem):
            pltpu.make_async_copy(x_hbm.at[pl.ds(0, C)], rbuf, sem).wait()

        def scatter_add(j, rbuf):
            @pl.loop(0, C // 16)
            def _(g):
                gg = (C // 16 - 1) - g
                pltpu.sync_copy(
                    rbuf.at[pl.ds(gg * 16, 16)],
                    acc.at[idx_d.at[j, pl.ds(gg * 16, 16)]],
                    add=True,
                )

        @pl.loop(0, 2)
        def _(h):
            pltpu.sync_copy(src_hbm.at[wid * 2 + h], idx_s)
            pltpu.sync_copy(dst_hbm.at[wid * 2 + h], idx_d)

            gather(0, r0, g0)

            @pl.loop(0, KH // 2)
            def _(jj):
                j0 = 2 * jj
                gather(j0 + 1, r1, g1)
                gather_wait(r0, g0)
                scatter_add(j0, r0)

                @pl.when(j0 + 2 < KH)
                def _():
                    gather(j0 + 2, r0, g0)

                gather_wait(r1, g1)
                scatter_add(j0 + 1, r1)

        plsc.subcore_barrier()

        pltpu.sync_copy(acc.at[pl.ds(sid * RPW, RPW)],
                        out_hbm.at[pl.ds(cid * ACC_ROWS + sid * RPW, RPW)])

    return segsum(x, src_r, dst_r, zeros)


def _mlp_body(x_ref, p0_ref, p1_ref, wa_ref, ba_ref, wb_ref, bb_ref, o_ref):
    pre = 1.1 * x_ref[...] + p0_ref[0] + p1_ref[0]
    t = _leaky(
        jnp.dot(pre, wa_ref[...], preferred_element_type=jnp.float32,
                precision=lax.Precision.HIGHEST) + ba_ref[...]
    )
    v = jnp.dot(t, wb_ref[...], preferred_element_type=jnp.float32,
                precision=lax.Precision.HIGHEST) + bb_ref[...]
    o_ref[...] = _leaky(_leaky(v))


def _mlp_layer(x, p, wa, ba, wb, bb):
    row = lambda i: (i, 0)
    full = lambda i: (0, 0)
    return pl.pallas_call(
        _mlp_body,
        out_shape=jax.ShapeDtypeStruct((N, D), jnp.float32),
        grid=(GRID,),
        in_specs=[
            pl.BlockSpec((BM, D), row),
            pl.BlockSpec((1, BM, D), lambda i: (0, i, 0)),
            pl.BlockSpec((1, BM, D), lambda i: (1, i, 0)),
            pl.BlockSpec((D, D), full),
            pl.BlockSpec((1, D), full),
            pl.BlockSpec((D, D), full),
            pl.BlockSpec((1, D), full),
        ],
        out_specs=pl.BlockSpec((BM, D), row),
    )(x, p, p, wa, ba, wb, bb)


def _mlp2_body(x_ref, p0_ref, p1_ref, wa_ref, ba_ref, wb_ref, bb_ref,
               wc1_ref, bc1_ref, wc2_ref, bc2_ref, o_ref, acc_ref):
    i = pl.program_id(0)
    pre = 1.1 * x_ref[...] + p0_ref[0] + p1_ref[0]
    t = _leaky(
        jnp.dot(pre, wa_ref[...], preferred_element_type=jnp.float32,
                precision=lax.Precision.HIGHEST) + ba_ref[...]
    )
    v = jnp.dot(t, wb_ref[...], preferred_element_type=jnp.float32,
                precision=lax.Precision.HIGHEST) + bb_ref[...]
    h2 = _leaky(_leaky(v))
    s = jnp.sum(h2, axis=0, keepdims=True)

    @pl.when(i == 0)
    def _():
        acc_ref[...] = s

    @pl.when(i > 0)
    def _():
        acc_ref[...] += s

    @pl.when(i == pl.num_programs(0) - 1)
    def _():
        em = acc_ref[...]
        z = _leaky(
            jnp.dot(em, wc1_ref[...], preferred_element_type=jnp.float32,
                    precision=lax.Precision.HIGHEST) + bc1_ref[...]
        )
        o_ref[...] = jnp.dot(
            z, wc2_ref[...], preferred_element_type=jnp.float32,
            precision=lax.Precision.HIGHEST) + bc2_ref[...]


def _mlp_layer2(h, p, wa, ba, wb, bb, wc1, bc1, wc2p, bc2p):
    row = lambda i: (i, 0)
    full = lambda i: (0, 0)
    return pl.pallas_call(
        _mlp2_body,
        out_shape=jax.ShapeDtypeStruct((1, D), jnp.float32),
        grid=(GRID,),
        in_specs=[
            pl.BlockSpec((BM, D), row),
            pl.BlockSpec((1, BM, D), lambda i: (0, i, 0)),
            pl.BlockSpec((1, BM, D), lambda i: (1, i, 0)),
            pl.BlockSpec((D, D), full),
            pl.BlockSpec((1, D), full),
            pl.BlockSpec((D, D), full),
            pl.BlockSpec((1, D), full),
            pl.BlockSpec((D, D), full),
            pl.BlockSpec((1, D), full),
            pl.BlockSpec((D, D), full),
            pl.BlockSpec((1, D), full),
        ],
        out_specs=pl.BlockSpec((1, D), full),
        scratch_shapes=[pltpu.VMEM((1, D), jnp.float32)],
    )(h, p, p, wa, ba, wb, bb, wc1, bc1, wc2p, bc2p)


def kernel(x, edge_index, W0a, b0a, W0b, b0b, W1a, b1a, W1b, b1b,
           Wc1, bc1, Wc2, bc2):
    src = edge_index[0].astype(jnp.int32).reshape(NW, EW)
    dst = edge_index[1].astype(jnp.int32).reshape(NW, EW)
    src_r = jnp.pad(src, ((0, 0), (0, EW_PAD - EW))).reshape(NW * 2, KH, C)
    dst_r = jnp.pad(dst, ((0, 0), (0, EW_PAD - EW)),
                    constant_values=TRASH).reshape(NW * 2, KH, C)
    zeros = jnp.zeros((RPW, D), jnp.float32)

    b0a_, b0b_, b1a_, b1b_, bc1_ = (
        b.reshape(1, D) for b in (b0a, b0b, b1a, b1b, bc1))
    Wc2p = jnp.pad(Wc2, ((0, 0), (0, D - Wc2.shape[1])))
    bc2p = jnp.pad(bc2, (0, D - bc2.shape[0])).reshape(1, D)

    p1 = _segment_sum_sc(x, src_r, dst_r, zeros).reshape(2, ACC_ROWS, D)
    h1 = _mlp_layer(x, p1, W0a, b0a_, W0b, b0b_)
    p2 = _segment_sum_sc(h1, src_r, dst_r, zeros).reshape(2, ACC_ROWS, D)
    cl = _mlp_layer2(h1, p2, W1a, b1a_, W1b, b1b_, Wc1, bc1_, Wc2p, bc2p)
    return cl[:, : Wc2.shape[1]]


# speedup vs baseline: 3.8559x; 1.0303x over previous
import functools

import jax
import jax.numpy as jnp
from jax import lax
from jax.experimental import pallas as pl
from jax.experimental.pallas import tpu as pltpu
from jax.experimental.pallas import tpu_sc as plsc

N = 10000
D = 128
E = 320000
NC = 2
NS = 16
NW = NC * NS
EW = E // NW
C = 128
K = 80
KH = K // 2
EW_PAD = K * C
RPW = 632
ACC_ROWS = NS * RPW
TRASH = N
BM = 1000
GRID = N // BM


def _leaky(v):
    return jnp.where(v > 0, v, 0.01 * v)


def _segment_sum_sc(x, src_r, dst_r, zeros):
    mesh = plsc.VectorSubcoreMesh(
        core_axis_name="c", subcore_axis_name="s", num_cores=NC, num_subcores=NS
    )

    @functools.partial(
        pl.kernel,
        out_type=jax.ShapeDtypeStruct((2 * ACC_ROWS, D), jnp.float32),
        mesh=mesh,
        scratch_types=[
            pltpu.VMEM((KH, C), jnp.int32),
            pltpu.VMEM((KH, C), jnp.int32),
            pltpu.VMEM((C, D), jnp.float32),
            pltpu.VMEM((C, D), jnp.float32),
            pltpu.VMEM_SHARED((ACC_ROWS, D), jnp.float32),
            pltpu.SemaphoreType.DMA,
            pltpu.SemaphoreType.DMA,
        ],
    )
    def segsum(x_hbm, src_hbm, dst_hbm, z_hbm, out_hbm,
               idx_s, idx_d, r0, r1, acc, g0, g1):
        cid = lax.axis_index("c")
        sid = lax.axis_index("s")
        wid = cid * NS + sid

        pltpu.sync_copy(z_hbm, acc.at[pl.ds(sid * RPW, RPW)])

        plsc.subcore_barrier()

        def gather(j, rbuf, sem):
            return pltpu.async_copy(x_hbm.at[idx_s.at[j]], rbuf, sem)

        def gather_wait(rbuf, sem):
            pltpu.make_async_copy(x_hbm.at[pl.ds(0, C)], rbuf, sem).wait()

        def scatter_add(j, rbuf):
            pltpu.sync_copy(rbuf, acc.at[idx_d.at[j]], add=True)

        @pl.loop(0, 2)
        def _(h):
            pltpu.sync_copy(src_hbm.at[wid * 2 + h], idx_s)
            pltpu.sync_copy(dst_hbm.at[wid * 2 + h], idx_d)

            gather(0, r0, g0)

            @pl.loop(0, KH // 2)
            def _(jj):
                j0 = 2 * jj
                gather(j0 + 1, r1, g1)
                gather_wait(r0, g0)
                scatter_add(j0, r0)

                @pl.when(j0 + 2 < KH)
                def _():
                    gather(j0 + 2, r0, g0)

                gather_wait(r1, g1)
                scatter_add(j0 + 1, r1)

        plsc.subcore_barrier()

        pltpu.sync_copy(acc.at[pl.ds(sid * RPW, RPW)],
                        out_hbm.at[pl.ds(cid * ACC_ROWS + sid * RPW, RPW)])

    return segsum(x, src_r, dst_r, zeros)


def _mlp_body(x_ref, p0_ref, p1_ref, wa_ref, ba_ref, wb_ref, bb_ref, o_ref):
    pre = 1.1 * x_ref[...] + p0_ref[0] + p1_ref[0]
    t = _leaky(
        jnp.dot(pre, wa_ref[...], preferred_element_type=jnp.float32,
                precision=lax.Precision.HIGHEST) + ba_ref[...]
    )
    v = jnp.dot(t, wb_ref[...], preferred_element_type=jnp.float32,
                precision=lax.Precision.HIGHEST) + bb_ref[...]
    o_ref[...] = _leaky(_leaky(v))


def _mlp_layer(x, p, wa, ba, wb, bb):
    row = lambda i: (i, 0)
    full = lambda i: (0, 0)
    return pl.pallas_call(
        _mlp_body,
        out_shape=jax.ShapeDtypeStruct((N, D), jnp.float32),
        grid=(GRID,),
        in_specs=[
            pl.BlockSpec((BM, D), row),
            pl.BlockSpec((1, BM, D), lambda i: (0, i, 0)),
            pl.BlockSpec((1, BM, D), lambda i: (1, i, 0)),
            pl.BlockSpec((D, D), full),
            pl.BlockSpec((1, D), full),
            pl.BlockSpec((D, D), full),
            pl.BlockSpec((1, D), full),
        ],
        out_specs=pl.BlockSpec((BM, D), row),
    )(x, p, p, wa, ba, wb, bb)


def _mlp2_body(x_ref, p0_ref, p1_ref, wa_ref, ba_ref, wb_ref, bb_ref,
               wc1_ref, bc1_ref, wc2_ref, bc2_ref, o_ref, acc_ref):
    i = pl.program_id(0)
    pre = 1.1 * x_ref[...] + p0_ref[0] + p1_ref[0]
    t = _leaky(
        jnp.dot(pre, wa_ref[...], preferred_element_type=jnp.float32,
                precision=lax.Precision.HIGHEST) + ba_ref[...]
    )
    v = jnp.dot(t, wb_ref[...], preferred_element_type=jnp.float32,
                precision=lax.Precision.HIGHEST) + bb_ref[...]
    h2 = _leaky(_leaky(v))
    s = jnp.sum(h2, axis=0, keepdims=True)

    @pl.when(i == 0)
    def _():
        acc_ref[...] = s

    @pl.when(i > 0)
    def _():
        acc_ref[...] += s

    @pl.when(i == pl.num_programs(0) - 1)
    def _():
        em = acc_ref[...]
        z = _leaky(
            jnp.dot(em, wc1_ref[...], preferred_element_type=jnp.float32,
                    precision=lax.Precision.HIGHEST) + bc1_ref[...]
        )
        o_ref[...] = jnp.dot(
            z, wc2_ref[...], preferred_element_type=jnp.float32,
            precision=lax.Precision.HIGHEST) + bc2_ref[...]


def _mlp_layer2(h, p, wa, ba, wb, bb, wc1, bc1, wc2p, bc2p):
    row = lambda i: (i, 0)
    full = lambda i: (0, 0)
    return pl.pallas_call(
        _mlp2_body,
        out_shape=jax.ShapeDtypeStruct((1, D), jnp.float32),
        grid=(GRID,),
        in_specs=[
            pl.BlockSpec((BM, D), row),
            pl.BlockSpec((1, BM, D), lambda i: (0, i, 0)),
            pl.BlockSpec((1, BM, D), lambda i: (1, i, 0)),
            pl.BlockSpec((D, D), full),
            pl.BlockSpec((1, D), full),
            pl.BlockSpec((D, D), full),
            pl.BlockSpec((1, D), full),
            pl.BlockSpec((D, D), full),
            pl.BlockSpec((1, D), full),
            pl.BlockSpec((D, D), full),
            pl.BlockSpec((1, D), full),
        ],
        out_specs=pl.BlockSpec((1, D), full),
        scratch_shapes=[pltpu.VMEM((1, D), jnp.float32)],
    )(h, p, p, wa, ba, wb, bb, wc1, bc1, wc2p, bc2p)


def kernel(x, edge_index, W0a, b0a, W0b, b0b, W1a, b1a, W1b, b1b,
           Wc1, bc1, Wc2, bc2):
    src = edge_index[0].astype(jnp.int32).reshape(NW, EW)
    dst = edge_index[1].astype(jnp.int32).reshape(NW, EW)
    src_r = jnp.pad(src, ((0, 0), (0, EW_PAD - EW))).reshape(NW * 2, KH, C)
    dst_r = jnp.pad(dst, ((0, 0), (0, EW_PAD - EW)),
                    constant_values=TRASH).reshape(NW * 2, KH, C)
    zeros = jnp.zeros((RPW, D), jnp.float32)

    b0a_, b0b_, b1a_, b1b_, bc1_ = (
        b.reshape(1, D) for b in (b0a, b0b, b1a, b1b, bc1))
    Wc2p = jnp.pad(Wc2, ((0, 0), (0, D - Wc2.shape[1])))
    bc2p = jnp.pad(bc2, (0, D - bc2.shape[0])).reshape(1, D)

    p1 = _segment_sum_sc(x, src_r, dst_r, zeros).reshape(2, ACC_ROWS, D)
    h1 = _mlp_layer(x, p1, W0a, b0a_, W0b, b0b_)
    p2 = _segment_sum_sc(h1, src_r, dst_r, zeros).reshape(2, ACC_ROWS, D)
    cl = _mlp_layer2(h1, p2, W1a, b1a_, W1b, b1b_, Wc1, bc1_, Wc2p, bc2p)
    return cl[:, : Wc2.shape[1]]
